# baseline (speedup 1.0000x reference)
import functools

import jax
import jax.numpy as jnp
from jax import lax
from jax.experimental import pallas as pl
from jax.experimental.pallas import tpu as pltpu
from jax.experimental.pallas import tpu_sc as plsc

_TOTAL = 8192
_DIM = 64
_NW = 32
_B = _TOTAL // _NW
_L = 16
_ENT = 1000000
_FULL_TC = _ENT // 128
_TAIL0 = _FULL_TC * 128
_WTC = 244
_CHUNK_TC = 4
_CW = _CHUNK_TC * 128
_OCAP = 1024
_K = 32
_TRASH = 2 * _TOTAL
_GSLOTS = 2 * _TOTAL + 1
_BSTRIDE = 137


def _rsqrt(x):
    xi = plsc.bitcast(x, jnp.int32)
    yi = jnp.int32(0x5F3759DF) - (xi >> 1)
    y = plsc.bitcast(yi, jnp.float32)
    for _ in range(3):
        y = y * (1.5 - 0.5 * x * y * y)
    return y


@jax.jit
def _extract(h, t, ent_t, ent_tail):
    @functools.partial(
        pl.kernel,
        mesh=plsc.VectorSubcoreMesh(core_axis_name="c", subcore_axis_name="s"),
        out_type=jax.ShapeDtypeStruct((_GSLOTS * 128,), jnp.float32),
        compiler_params=pltpu.CompilerParams(needs_layout_passes=False),
        scratch_types=[
            pltpu.VMEM((_TOTAL,), jnp.int32),
            pltpu.VMEM((_TOTAL,), jnp.int32),
            pltpu.VMEM((_OCAP,), jnp.int32),
            pltpu.VMEM((_OCAP,), jnp.int32),
            pltpu.VMEM((_OCAP,), jnp.int32),
            pltpu.VMEM((_OCAP,), jnp.int32),
            pltpu.VMEM((_DIM, _CW), jnp.float32),
            pltpu.VMEM((_DIM, _CW), jnp.float32),
            pltpu.VMEM((_K * 128,), jnp.float32),
            pltpu.SemaphoreType.DMA,
            pltpu.SemaphoreType.DMA,
            pltpu.SemaphoreType.DMA,
        ],
    )
    def k(h_hbm, t_hbm, ent_hbm, tail_hbm, g_hbm,
          hbuf, tbuf, oid, oslot, cid, cslot, slab0, slab1, stg,
          sem0, sem1, semg):
        wid = lax.axis_index("s") * 2 + lax.axis_index("c")
        is_last = wid == _NW - 1
        base_tc = wid * _WTC
        glo = base_tc * 128
        ghi = jnp.where(is_last, _ENT, glo + _WTC * 128)
        nchunks = jnp.where(is_last, 62, 61)

        pltpu.sync_copy(h_hbm, hbuf)
        pltpu.sync_copy(t_hbm, tbuf)

        lanes = lax.iota(jnp.int32, _L)
        zero_i = jnp.zeros((_L,), jnp.int32)

        @pl.loop(0, _OCAP // _L)
        def _z(v):
            cid[pl.ds(v * _L, _L)] = zero_i
            cslot[pl.ds(v * _L, _L)] = zero_i

        def scan_one(buf, slot_off):
            def body(v, cnt):
                ids = buf[pl.ds(v * _L, _L)]
                m = (ids >= glo) & (ids < ghi)
                plsc.store_compressed(oid.at[pl.ds(cnt, _L)], ids, mask=m)
                plsc.store_compressed(
                    oslot.at[pl.ds(cnt, _L)],
                    slot_off + v * _L + lanes, mask=m)
                npop = plsc.all_reduce_population_count(m)
                return cnt + npop[0]
            return body

        cnt = lax.fori_loop(0, _TOTAL // _L, scan_one(hbuf, 0), 0)
        cnt = lax.fori_loop(0, _TOTAL // _L, scan_one(tbuf, _TOTAL), cnt)

        def extract_region(e0, width, slab):
            def fbody(v, kcnt):
                ids = oid[pl.ds(v * _L, _L)]
                slots = oslot[pl.ds(v * _L, _L)]
                valid = (v * _L + lanes) < cnt
                m = valid & (ids >= e0) & (ids < e0 + width)
                plsc.store_compressed(cid.at[pl.ds(kcnt, _L)], ids - e0, mask=m)
                plsc.store_compressed(cslot.at[pl.ds(kcnt, _L)], slots, mask=m)
                npop = plsc.all_reduce_population_count(m)
                return kcnt + npop[0]

            kcnt = lax.fori_loop(0, (cnt + _L - 1) // _L, fbody, 0)

            for i in range(_K):
                if i % _L == 0:
                    cvec = cid[pl.ds(i, _L)]
                    svec = cslot[pl.ds(i, _L)]
                valid = i < kcnt
                lc = jnp.where(valid, cvec[i % _L], 0)
                gslot = jnp.where(valid, svec[i % _L], _TRASH)
                for c in range(_DIM // _L):
                    col = plsc.load_gather(
                        slab, [lanes + c * _L, jnp.full((_L,), lc, jnp.int32)])
                    stg[pl.ds(i * 128 + c * _L, _L)] = col
                pltpu.async_copy(
                    stg.at[pl.ds(i * 128, 128)],
                    g_hbm.at[pl.ds(pl.multiple_of(gslot * 128, 128), 128)],
                    semg)
            pltpu.make_async_copy(
                ent_hbm.at[0, pl.ds(0, _K * 128)],
                stg, semg).wait()

        def fire(cidx, slab, sem):
            e0 = pl.multiple_of((base_tc + cidx * _CHUNK_TC) * 128, 128)
            pltpu.async_copy(ent_hbm.at[:, pl.ds(e0, _CW)], slab, sem)

        def drain_slab(slab, sem):
            pltpu.make_async_copy(
                ent_hbm.at[:, pl.ds(0, _CW)], slab, sem).wait()

        fire(0, slab0, sem0)

        @pl.loop(0, 62)
        def _chunk(ci):
            @pl.when(ci < nchunks)
            def _():
                nxt = ci + 1

                @pl.when((nxt < nchunks) & (nxt % 2 == 0))
                def _():
                    fire(nxt, slab0, sem0)

                @pl.when((nxt < nchunks) & (nxt % 2 == 1))
                def _():
                    fire(nxt, slab1, sem1)

                e0 = (base_tc + ci * _CHUNK_TC) * 128

                @pl.when(ci % 2 == 0)
                def _():
                    drain_slab(slab0, sem0)
                    extract_region(e0, _CW, slab0)

                @pl.when(ci % 2 == 1)
                def _():
                    drain_slab(slab1, sem1)
                    extract_region(e0, _CW, slab1)

        @pl.when(is_last)
        def _():
            pltpu.sync_copy(tail_hbm, slab0.at[:, pl.ds(0, 128)])
            extract_region(_TAIL0, 64, slab0)

    return k(h, t, ent_t, ent_tail)


@jax.jit
def _score(r, rel_pad, g):
    @functools.partial(
        pl.kernel,
        mesh=plsc.VectorSubcoreMesh(core_axis_name="c", subcore_axis_name="s"),
        out_type=jax.ShapeDtypeStruct((_TOTAL,), jnp.float32),
        compiler_params=pltpu.CompilerParams(needs_layout_passes=False),
        scratch_types=[
            pltpu.VMEM((_B,), jnp.int32),
            pltpu.VMEM((_DIM, 512), jnp.float32),
            pltpu.VMEM((_B * _DIM,), jnp.float32),
            pltpu.VMEM((64 * 128,), jnp.float32),
            pltpu.VMEM((64 * 128,), jnp.float32),
            pltpu.VMEM((6 * _L * _BSTRIDE,), jnp.float32),
            pltpu.VMEM((_B,), jnp.float32),
        ],
    )
    def k(r_hbm, rel_hbm, g_hbm, out_hbm, rbuf, rslab, rcols, hgb, tgb,
          bounce, sc):
        wid = lax.axis_index("s") * 2 + lax.axis_index("c")
        base = wid * _B
        pltpu.sync_copy(r_hbm.at[pl.ds(base, _B)], rbuf)

        lanes = lax.iota(jnp.int32, _L)
        zero = jnp.zeros((_L,), jnp.float32)

        @pl.loop(0, 2)
        def _round(rnd):
            pltpu.sync_copy(
                rel_hbm.at[:, pl.ds(pl.multiple_of(rnd * 512, 128), 512)],
                rslab)

            @pl.loop(0, _B // _L)
            def _wave(wv):
                rids = rbuf[pl.ds(wv * _L, _L)]
                for j in range(_L):
                    rid = rids[j]

                    @pl.when((rid >> 9) == rnd)
                    def _():
                        lc = rid - rnd * 512
                        for c in range(_DIM // _L):
                            col = plsc.load_gather(
                                rslab,
                                [lanes + c * _L,
                                 jnp.full((_L,), lc, jnp.int32)])
                            rcols[pl.ds((wv * _L + j) * _DIM + c * _L,
                                        _L)] = col

        @pl.loop(0, 4)
        def _sub(s):
            goff = pl.multiple_of((base + s * 64) * 128, 128)
            pltpu.sync_copy(g_hbm.at[pl.ds(goff, 64 * 128)], hgb)
            goff_t = pl.multiple_of((_TOTAL + base + s * 64) * 128, 128)
            pltpu.sync_copy(g_hbm.at[pl.ds(goff_t, 64 * 128)], tgb)

            @pl.loop(0, 4)
            def _wave(wv):
                for j in range(_L):
                    lt = wv * _L + j
                    hv = [hgb[pl.ds(lt * 128 + c * _L, _L)] for c in range(4)]
                    tv = [tgb[pl.ds(lt * 128 + c * _L, _L)] for c in range(4)]
                    rbase = (s * 64 + wv * _L + j) * _DIM
                    rv = [rcols[pl.ds(rbase + c * _L, _L)] for c in range(4)]
                    parts = [zero] * 6
                    for c in range(4):
                        parts[0] = parts[0] + hv[c] * hv[c]
                        parts[1] = parts[1] + rv[c] * rv[c]
                        parts[2] = parts[2] + tv[c] * tv[c]
                        parts[3] = parts[3] + hv[c] * rv[c]
                        parts[4] = parts[4] + hv[c] * tv[c]
                        parts[5] = parts[5] + rv[c] * tv[c]
                    for d in range(6):
                        plsc.store_scatter(
                            bounce,
                            [jnp.full((_L,),
                                      d * _L * _BSTRIDE + j * _BSTRIDE,
                                      jnp.int32) + lanes],
                            parts[d])
                dots = []
                for d in range(6):
                    db = d * _L * _BSTRIDE
                    acc = zero
                    for l in range(_L):
                        acc = acc + plsc.load_gather(
                            bounce, [lanes * _BSTRIDE + (db + l)])
                    dots.append(acc)
                vhh, vrr, vtt, vhr, vht, vrt = dots
                s2 = 3.0 + 2.0 * (vhr * _rsqrt(vhh * vrr)
                                  - vht * _rsqrt(vhh * vtt)
                                  - vrt * _rsqrt(vrr * vtt))
                s2 = jnp.maximum(s2, 0.0)
                sc[pl.ds((s * 4 + wv) * _L, _L)] = (
                    s2 * _rsqrt(jnp.maximum(s2, 1e-20)))

        pltpu.sync_copy(sc, out_hbm.at[pl.ds(base, _B)])

    return k(r, rel_pad, g)


def kernel(h, r, t, ent_emb, rel_emb):
    h = h.astype(jnp.int32)
    r = r.astype(jnp.int32)
    t = t.astype(jnp.int32)
    ent_t = ent_emb.T
    ent_tail = jnp.pad(ent_emb[_TAIL0:].T, ((0, 0), (0, 128 - (_ENT - _TAIL0))))
    rel_pad = jnp.pad(rel_emb.T, ((0, 0), (0, 1024 - rel_emb.shape[0])))
    g = _extract(h, t, ent_t, ent_tail)
    score = _score(r, rel_pad, g)
    bs = _TOTAL // 2
    p_score = score[:bs].reshape(1, bs).transpose(1, 0)
    n_score = score[bs:].reshape(1, bs).transpose(1, 0)
    return (p_score, n_score)


# speedup vs baseline: 3.4415x; 3.4415x over previous
import functools

import jax
import jax.numpy as jnp
from jax import lax
from jax.experimental import pallas as pl
from jax.experimental.pallas import tpu as pltpu
from jax.experimental.pallas import tpu_sc as plsc

_TOTAL = 8192
_DIM = 64
_NW = 32
_B = _TOTAL // _NW
_L = 16


def _rsqrt(x):
    xi = plsc.bitcast(x, jnp.int32)
    yi = jnp.int32(0x5F3759DF) - (xi >> 1)
    y = plsc.bitcast(yi, jnp.float32)
    for _ in range(3):
        y = y * (1.5 - 0.5 * x * y * y)
    return y


@jax.jit
def _scores(h, r, t, ent2, rel_emb):
    @functools.partial(
        pl.kernel,
        mesh=plsc.VectorSubcoreMesh(core_axis_name="c", subcore_axis_name="s"),
        out_type=jax.ShapeDtypeStruct((_TOTAL,), jnp.float32),
        compiler_params=pltpu.CompilerParams(
            needs_layout_passes=False, use_tc_tiling_on_sc=False),
        scratch_types=[
            pltpu.VMEM((_B,), jnp.int32),
            pltpu.VMEM((_B,), jnp.int32),
            pltpu.VMEM((_B,), jnp.int32),
            pltpu.VMEM((_B,), jnp.int32),
            pltpu.VMEM((_B,), jnp.int32),
            pltpu.VMEM((_B, 2 * _DIM), jnp.float32),
            pltpu.VMEM((_B, 2 * _DIM), jnp.float32),
            pltpu.VMEM((_B, _DIM), jnp.float32),
            pltpu.VMEM((_B,), jnp.float32),
            pltpu.SemaphoreType.DMA,
        ],
    )
    def k(h_hbm, r_hbm, t_hbm, ent_hbm, rel_hbm, out_hbm,
          hi, ri, ti, hi2, ti2, hrow, trow, rrow, sc, sem):
        wid = lax.axis_index("s") * 2 + lax.axis_index("c")
        base = wid * _B
        pltpu.sync_copy(h_hbm.at[pl.ds(base, _B)], hi)
        pltpu.sync_copy(r_hbm.at[pl.ds(base, _B)], ri)
        pltpu.sync_copy(t_hbm.at[pl.ds(base, _B)], ti)

        @pl.loop(0, _B // _L)
        def _shift(g):
            b = g * _L
            hi2[pl.ds(b, _L)] = hi[pl.ds(b, _L)] >> 1
            ti2[pl.ds(b, _L)] = ti[pl.ds(b, _L)] >> 1

        cph = pltpu.async_copy(ent_hbm.at[hi2], hrow, sem)
        cpr = pltpu.async_copy(rel_hbm.at[ri], rrow, sem)
        cpt = pltpu.async_copy(ent_hbm.at[ti2], trow, sem)
        cph.wait()
        cpr.wait()
        cpt.wait()

        lanes = lax.iota(jnp.int32, _L)
        zero = jnp.zeros((_L,), jnp.float32)

        @pl.loop(0, _B // _L)
        def _grp(g):
            b = g * _L
            rows = b + lanes
            hoff = (hi[pl.ds(b, _L)] & 1) * _DIM
            toff = (ti[pl.ds(b, _L)] & 1) * _DIM
            vhh = vrr = vtt = vhr = vht = vrt = zero
            for c in range(_DIM):
                col = jnp.full((_L,), c, jnp.int32)
                hc = plsc.load_gather(hrow, [rows, hoff + c])
                rc = plsc.load_gather(rrow, [rows, col])
                tc = plsc.load_gather(trow, [rows, toff + c])
                vhh = vhh + hc * hc
                vrr = vrr + rc * rc
                vtt = vtt + tc * tc
                vhr = vhr + hc * rc
                vht = vht + hc * tc
                vrt = vrt + rc * tc
            s2 = 3.0 + 2.0 * (vhr * _rsqrt(vhh * vrr)
                              - vht * _rsqrt(vhh * vtt)
                              - vrt * _rsqrt(vrr * vtt))
            s2 = jnp.maximum(s2, 0.0)
            sc[pl.ds(b, _L)] = s2 * _rsqrt(jnp.maximum(s2, 1e-20))

        pltpu.sync_copy(sc, out_hbm.at[pl.ds(base, _B)])

    return k(h, r, t, ent2, rel_emb)


def kernel(h, r, t, ent_emb, rel_emb):
    h = h.astype(jnp.int32)
    r = r.astype(jnp.int32)
    t = t.astype(jnp.int32)
    ent2 = ent_emb.reshape(ent_emb.shape[0] // 2, 2 * _DIM)
    score = _scores(h, r, t, ent2, rel_emb)
    bs = _TOTAL // 2
    p_score = score[:bs].reshape(1, bs).transpose(1, 0)
    n_score = score[bs:].reshape(1, bs).transpose(1, 0)
    return (p_score, n_score)
